# baseline (speedup 1.0000x reference)
import functools

import jax
import jax.numpy as jnp
from jax import lax
from jax.experimental import pallas as pl
from jax.experimental.pallas import tpu as pltpu
from jax.experimental.pallas import tpu_sc as plsc

B = 16384
NE = 1000000
NR = 1000
D = 64
W = 2 * D
L = 16
NC = 2
NS = 16
NW = NC * NS
BPW = B // NW
CHUNK = 128
NCHUNK = BPW // CHUNK
EB = 256
NB_FULL = NE // EB
NE_TAIL = NE - NB_FULL * EB
BLOCKS_PER_W = (NB_FULL + 1 + NW - 1) // NW


KMAIN = NB_FULL // NW


def _format_sc(entT_hbm, tail_hbm, fmt_hbm,
               in0, in1, out0, out1, si0, si1, so0, so1):
    wid = lax.axis_index("s") * NC + lax.axis_index("c")
    lane = lax.iota(jnp.int32, L)
    parb = (lane & 1) * D
    halfrow = lane >> 1

    def transpose_groups(buf_in, buf_out, n_groups):
        def dim_body(d, carry):
            col = ((lane + d) & (D - 1)) + parb
            for g in range(n_groups):
                vals = buf_in[d, pl.ds(g * L, L)]
                plsc.store_scatter(buf_out, [g * (L // 2) + halfrow, col], vals)
            return carry
        lax.fori_loop(0, D, dim_body, 0, unroll=8)

    def start_in(k, buf, sem):
        c = k * NW + wid
        pltpu.async_copy(entT_hbm.at[:, pl.ds(c * EB, EB)], buf, sem)

    def wait_in(buf, sem):
        pltpu.make_async_copy(entT_hbm.at[:, pl.ds(0, EB)], buf, sem).wait()

    def start_out(k, buf, sem):
        c = k * NW + wid
        pltpu.async_copy(buf, fmt_hbm.at[pl.ds(c * (EB // 2), EB // 2), :], sem)

    def wait_out(buf, sem):
        pltpu.make_async_copy(
            buf, fmt_hbm.at[pl.ds(0, EB // 2), :], sem).wait()

    start_in(0, in0, si0)
    start_in(1, in1, si1)

    def pair_body(j, carry):
        k0 = 2 * j
        wait_in(in0, si0)

        @pl.when(j > 0)
        def _():
            wait_out(out0, so0)

        transpose_groups(in0, out0, EB // L)
        start_out(k0, out0, so0)

        @pl.when(k0 + 2 < KMAIN)
        def _():
            start_in(k0 + 2, in0, si0)

        wait_in(in1, si1)

        @pl.when(j > 0)
        def _():
            wait_out(out1, so1)

        transpose_groups(in1, out1, EB // L)
        start_out(k0 + 1, out1, so1)

        @pl.when(k0 + 3 < KMAIN)
        def _():
            start_in(k0 + 3, in1, si1)

        return carry

    lax.fori_loop(0, KMAIN // 2, pair_body, 0)
    wait_out(out0, so0)
    wait_out(out1, so1)

    c_last = KMAIN * NW + wid

    @pl.when(c_last < NB_FULL)
    def _full():
        pltpu.sync_copy(entT_hbm.at[:, pl.ds(c_last * EB, EB)], in0)
        transpose_groups(in0, out0, EB // L)
        pltpu.sync_copy(out0, fmt_hbm.at[pl.ds(c_last * (EB // 2), EB // 2), :])

    @pl.when(c_last == NB_FULL)
    def _tail():
        pltpu.sync_copy(tail_hbm, in0)
        transpose_groups(in0, out0, NE_TAIL // L)
        pltpu.sync_copy(out0.at[pl.ds(0, NE_TAIL // 2), :],
                        fmt_hbm.at[pl.ds(NB_FULL * (EB // 2), NE_TAIL // 2), :])


def _score_sc(ph_hbm, pr_hbm, pt_hbm, nh_hbm, nt_hbm, ent_hbm, rel_hbm,
              pos_hbm, neg_hbm,
              phv, prv, ptv, nhv, ntv,
              phh, pth, nhh, nth,
              ph_rows, pt_rows, nh_rows, nt_rows, r_rows,
              pos_v, neg_v, sem):
    wid = lax.axis_index("s") * NC + lax.axis_index("c")
    lane = lax.iota(jnp.int32, L)

    def chunk_body(c, chunk_carry):
        base = wid * BPW + c * CHUNK
        sl = pl.ds(base, CHUNK)
        pltpu.sync_copy(ph_hbm.at[sl], phv)
        pltpu.sync_copy(pr_hbm.at[sl], prv)
        pltpu.sync_copy(pt_hbm.at[sl], ptv)
        pltpu.sync_copy(nh_hbm.at[sl], nhv)
        pltpu.sync_copy(nt_hbm.at[sl], ntv)

        def halve(i, carry):
            ds16 = pl.ds(i * L, L)
            phh[ds16] = phv[ds16] >> 1
            pth[ds16] = ptv[ds16] >> 1
            nhh[ds16] = nhv[ds16] >> 1
            nth[ds16] = ntv[ds16] >> 1
            return carry

        lax.fori_loop(0, CHUNK // L, halve, 0)

        g1 = pltpu.async_copy(ent_hbm.at[phh], ph_rows, sem)
        g2 = pltpu.async_copy(ent_hbm.at[pth], pt_rows, sem)
        g3 = pltpu.async_copy(ent_hbm.at[nhh], nh_rows, sem)
        g4 = pltpu.async_copy(ent_hbm.at[nth], nt_rows, sem)
        g5 = pltpu.async_copy(rel_hbm.at[prv], r_rows, sem)
        g1.wait(); g2.wait(); g3.wait(); g4.wait(); g5.wait()

        def body(g, carry):
            ds16 = pl.ds(g * L, L)
            rowidx = g * L + lane
            phi = phv[ds16]
            pti = ptv[ds16]
            nhi = nhv[ds16]
            nti = ntv[ds16]
            pha = lane + (phi & (L - 1))
            pta = lane + (pti & (L - 1))
            nha = lane + (nhi & (L - 1))
            nta = lane + (nti & (L - 1))
            phb = (phi & 1) * D
            ptb = (pti & 1) * D
            nhb = (nhi & 1) * D
            ntb = (nti & 1) * D
            pacc = jnp.zeros((L,), jnp.float32)
            nacc = jnp.zeros((L,), jnp.float32)
            for d in range(D):
                rcol = (lane + d) & (D - 1)
                r = plsc.load_gather(r_rows, [rowidx, rcol])
                ph = plsc.load_gather(ph_rows, [rowidx, ((pha + d) & (D - 1)) + phb])
                pt = plsc.load_gather(pt_rows, [rowidx, ((pta + d) & (D - 1)) + ptb])
                nh = plsc.load_gather(nh_rows, [rowidx, ((nha + d) & (D - 1)) + nhb])
                nt = plsc.load_gather(nt_rows, [rowidx, ((nta + d) & (D - 1)) + ntb])
                pacc = pacc + jnp.abs(ph + r - pt)
                nacc = nacc + jnp.abs(nh + r - nt)
            pos_v[pl.ds(g * L, L)] = pacc
            neg_v[pl.ds(g * L, L)] = nacc
            return carry

        lax.fori_loop(0, CHUNK // L, body, 0)
        pltpu.sync_copy(pos_v, pos_hbm.at[sl])
        pltpu.sync_copy(neg_v, neg_hbm.at[sl])
        return chunk_carry

    lax.fori_loop(0, NCHUNK, chunk_body, 0)


@jax.jit
def kernel(pos_samples, neg_samples, entity_table, relation_table):
    ph = pos_samples[:, 0].astype(jnp.int32)
    pr = pos_samples[:, 1].astype(jnp.int32)
    pt = pos_samples[:, 2].astype(jnp.int32)
    nh = neg_samples[:, 0].astype(jnp.int32)
    nt = neg_samples[:, 2].astype(jnp.int32)
    entT = entity_table.T
    tailp = jnp.pad(entT[:, NB_FULL * EB:], ((0, 0), (0, EB - NE_TAIL)))
    relp = jnp.pad(relation_table, ((0, 0), (0, W - D)))

    mesh = plsc.VectorSubcoreMesh(core_axis_name="c", subcore_axis_name="s")
    params = pltpu.CompilerParams(
        needs_layout_passes=False, use_tc_tiling_on_sc=True)

    fmt = pl.kernel(
        _format_sc,
        out_type=jax.ShapeDtypeStruct((NE // 2, W), jnp.float32),
        mesh=mesh,
        compiler_params=params,
        scratch_types=[
            pltpu.VMEM((D, EB), jnp.float32),
            pltpu.VMEM((D, EB), jnp.float32),
            pltpu.VMEM((EB // 2, W), jnp.float32),
            pltpu.VMEM((EB // 2, W), jnp.float32),
            pltpu.SemaphoreType.DMA,
            pltpu.SemaphoreType.DMA,
            pltpu.SemaphoreType.DMA,
            pltpu.SemaphoreType.DMA,
        ],
    )(entT, tailp)

    score = pl.kernel(
        _score_sc,
        out_type=(
            jax.ShapeDtypeStruct((B,), jnp.float32),
            jax.ShapeDtypeStruct((B,), jnp.float32),
        ),
        mesh=mesh,
        compiler_params=params,
        scratch_types=[
            pltpu.VMEM((CHUNK,), jnp.int32),
            pltpu.VMEM((CHUNK,), jnp.int32),
            pltpu.VMEM((CHUNK,), jnp.int32),
            pltpu.VMEM((CHUNK,), jnp.int32),
            pltpu.VMEM((CHUNK,), jnp.int32),
            pltpu.VMEM((CHUNK,), jnp.int32),
            pltpu.VMEM((CHUNK,), jnp.int32),
            pltpu.VMEM((CHUNK,), jnp.int32),
            pltpu.VMEM((CHUNK,), jnp.int32),
            pltpu.VMEM((CHUNK, W), jnp.float32),
            pltpu.VMEM((CHUNK, W), jnp.float32),
            pltpu.VMEM((CHUNK, W), jnp.float32),
            pltpu.VMEM((CHUNK, W), jnp.float32),
            pltpu.VMEM((CHUNK, W), jnp.float32),
            pltpu.VMEM((CHUNK,), jnp.float32),
            pltpu.VMEM((CHUNK,), jnp.float32),
            pltpu.SemaphoreType.DMA,
        ],
    )
    return score(ph, pr, pt, nh, nt, fmt, relp)


# speedup vs baseline: 1.0046x; 1.0046x over previous
import jax
import jax.numpy as jnp
from jax import lax
from jax.experimental import pallas as pl
from jax.experimental.pallas import tpu as pltpu
from jax.experimental.pallas import tpu_sc as plsc

B = 16384
NE = 1000000
NR = 1000
D = 64
W = 2 * D
L = 16
NC = 2
NS = 16
NW = NC * NS
BPW = B // NW
CHUNK = 128
NCHUNK = BPW // CHUNK

EB = 2048
GRID = (NE + EB - 1) // EB
FR = GRID * (EB // 2)


def _format_tc(x_ref, o_ref):
    for k in range(EB // 128):
        t = jnp.transpose(x_ref[:, k * 128:(k + 1) * 128])
        r0 = (k // 2) * 128
        c0 = (k % 2) * D
        o_ref[r0:r0 + 128, c0:c0 + D] = t


def _score_sc(ph_hbm, pr_hbm, pt_hbm, nh_hbm, nt_hbm, ent_hbm, rel_hbm,
              pos_hbm, neg_hbm,
              phv, prv, ptv, nhv, ntv,
              phh, pth, nhh, nth,
              ph_rows, pt_rows, nh_rows, nt_rows, r_rows,
              pos_v, neg_v, sem):
    wid = lax.axis_index("s") * NC + lax.axis_index("c")
    lane = lax.iota(jnp.int32, L)

    def chunk_body(c, chunk_carry):
        base = wid * BPW + c * CHUNK
        sl = pl.ds(base, CHUNK)
        pltpu.sync_copy(ph_hbm.at[sl], phv)
        pltpu.sync_copy(pr_hbm.at[sl], prv)
        pltpu.sync_copy(pt_hbm.at[sl], ptv)
        pltpu.sync_copy(nh_hbm.at[sl], nhv)
        pltpu.sync_copy(nt_hbm.at[sl], ntv)

        def rowmap(i, carry):
            ds16 = pl.ds(i * L, L)
            phh[ds16] = ((phv[ds16] >> 8) << 7) + (phv[ds16] & 127)
            pth[ds16] = ((ptv[ds16] >> 8) << 7) + (ptv[ds16] & 127)
            nhh[ds16] = ((nhv[ds16] >> 8) << 7) + (nhv[ds16] & 127)
            nth[ds16] = ((ntv[ds16] >> 8) << 7) + (ntv[ds16] & 127)
            return carry

        lax.fori_loop(0, CHUNK // L, rowmap, 0)

        g1 = pltpu.async_copy(ent_hbm.at[phh], ph_rows, sem)
        g2 = pltpu.async_copy(ent_hbm.at[pth], pt_rows, sem)
        g3 = pltpu.async_copy(ent_hbm.at[nhh], nh_rows, sem)
        g4 = pltpu.async_copy(ent_hbm.at[nth], nt_rows, sem)
        g5 = pltpu.async_copy(rel_hbm.at[prv], r_rows, sem)
        g1.wait(); g2.wait(); g3.wait(); g4.wait(); g5.wait()

        def body(g, carry):
            ds16 = pl.ds(g * L, L)
            rowidx = g * L + lane
            phi = phv[ds16]
            pti = ptv[ds16]
            nhi = nhv[ds16]
            nti = ntv[ds16]
            phb = ((phi >> 7) & 1) * D
            ptb = ((pti >> 7) & 1) * D
            nhb = ((nhi >> 7) & 1) * D
            ntb = ((nti >> 7) & 1) * D
            pacc = jnp.zeros((L,), jnp.float32)
            nacc = jnp.zeros((L,), jnp.float32)
            for d in range(D):
                rcol = (lane + d) & (D - 1)
                r = plsc.load_gather(r_rows, [rowidx, rcol])
                ph = plsc.load_gather(ph_rows, [rowidx, rcol + phb])
                pt = plsc.load_gather(pt_rows, [rowidx, rcol + ptb])
                nh = plsc.load_gather(nh_rows, [rowidx, rcol + nhb])
                nt = plsc.load_gather(nt_rows, [rowidx, rcol + ntb])
                pacc = pacc + jnp.abs(ph + r - pt)
                nacc = nacc + jnp.abs(nh + r - nt)
            pos_v[pl.ds(g * L, L)] = pacc
            neg_v[pl.ds(g * L, L)] = nacc
            return carry

        lax.fori_loop(0, CHUNK // L, body, 0)
        pltpu.sync_copy(pos_v, pos_hbm.at[sl])
        pltpu.sync_copy(neg_v, neg_hbm.at[sl])
        return chunk_carry

    lax.fori_loop(0, NCHUNK, chunk_body, 0)


@jax.jit
def kernel(pos_samples, neg_samples, entity_table, relation_table):
    ph = pos_samples[:, 0].astype(jnp.int32)
    pr = pos_samples[:, 1].astype(jnp.int32)
    pt = pos_samples[:, 2].astype(jnp.int32)
    nh = neg_samples[:, 0].astype(jnp.int32)
    nt = neg_samples[:, 2].astype(jnp.int32)
    entT = entity_table.T
    relp = jnp.pad(relation_table, ((0, 0), (0, W - D)))

    fmt = pl.pallas_call(
        _format_tc,
        grid=(GRID,),
        in_specs=[pl.BlockSpec((D, EB), lambda j: (0, j))],
        out_specs=pl.BlockSpec((EB // 2, W), lambda j: (j, 0)),
        out_shape=jax.ShapeDtypeStruct((FR, W), jnp.float32),
        compiler_params=pltpu.CompilerParams(
            dimension_semantics=("arbitrary",)),
    )(entT)

    mesh = plsc.VectorSubcoreMesh(core_axis_name="c", subcore_axis_name="s")
    params = pltpu.CompilerParams(
        needs_layout_passes=False, use_tc_tiling_on_sc=True)

    score = pl.kernel(
        _score_sc,
        out_type=(
            jax.ShapeDtypeStruct((B,), jnp.float32),
            jax.ShapeDtypeStruct((B,), jnp.float32),
        ),
        mesh=mesh,
        compiler_params=params,
        scratch_types=[
            pltpu.VMEM((CHUNK,), jnp.int32),
            pltpu.VMEM((CHUNK,), jnp.int32),
            pltpu.VMEM((CHUNK,), jnp.int32),
            pltpu.VMEM((CHUNK,), jnp.int32),
            pltpu.VMEM((CHUNK,), jnp.int32),
            pltpu.VMEM((CHUNK,), jnp.int32),
            pltpu.VMEM((CHUNK,), jnp.int32),
            pltpu.VMEM((CHUNK,), jnp.int32),
            pltpu.VMEM((CHUNK,), jnp.int32),
            pltpu.VMEM((CHUNK, W), jnp.float32),
            pltpu.VMEM((CHUNK, W), jnp.float32),
            pltpu.VMEM((CHUNK, W), jnp.float32),
            pltpu.VMEM((CHUNK, W), jnp.float32),
            pltpu.VMEM((CHUNK, W), jnp.float32),
            pltpu.VMEM((CHUNK,), jnp.float32),
            pltpu.VMEM((CHUNK,), jnp.float32),
            pltpu.SemaphoreType.DMA,
        ],
    )
    return score(ph, pr, pt, nh, nt, fmt, relp)


# speedup vs baseline: 1.1454x; 1.1401x over previous
import jax
import jax.numpy as jnp
from jax import lax
from jax.experimental import pallas as pl
from jax.experimental.pallas import tpu as pltpu
from jax.experimental.pallas import tpu_sc as plsc

B = 16384
NE = 1000000
NR = 1000
D = 64
W = 2 * D
L = 16
NC = 2
NS = 16
NW = NC * NS
BPW = B // NW
CHUNK = 128
NCHUNK = BPW // CHUNK

EB = 2048
GRID = (NE + EB - 1) // EB
FR = GRID * (EB // 2)


def _format_tc(x_ref, o_ref):
    for m in range(EB // 256):
        xa = x_ref[:, m * 256:m * 256 + 128]
        xb = x_ref[:, m * 256 + 128:m * 256 + 256]
        x2 = jnp.concatenate([xa, xb], axis=0)
        o_ref[m * 128:(m + 1) * 128, :] = jnp.transpose(x2)


def _score_sc(ph_hbm, pr_hbm, pt_hbm, nh_hbm, nt_hbm, ent_hbm, rel_hbm,
              pos_hbm, neg_hbm,
              phv, prv, ptv, nhv, ntv,
              phh, pth, nhh, nth,
              ph_rows, pt_rows, nh_rows, nt_rows, r_rows,
              pos_v, neg_v, sem):
    wid = lax.axis_index("s") * NC + lax.axis_index("c")
    lane = lax.iota(jnp.int32, L)

    def chunk_body(c, chunk_carry):
        base = wid * BPW + c * CHUNK
        sl = pl.ds(base, CHUNK)
        pltpu.sync_copy(ph_hbm.at[sl], phv)
        pltpu.sync_copy(pr_hbm.at[sl], prv)
        pltpu.sync_copy(pt_hbm.at[sl], ptv)
        pltpu.sync_copy(nh_hbm.at[sl], nhv)
        pltpu.sync_copy(nt_hbm.at[sl], ntv)

        def rowmap(i, carry):
            ds16 = pl.ds(i * L, L)
            phh[ds16] = ((phv[ds16] >> 8) << 7) + (phv[ds16] & 127)
            pth[ds16] = ((ptv[ds16] >> 8) << 7) + (ptv[ds16] & 127)
            nhh[ds16] = ((nhv[ds16] >> 8) << 7) + (nhv[ds16] & 127)
            nth[ds16] = ((ntv[ds16] >> 8) << 7) + (ntv[ds16] & 127)
            return carry

        lax.fori_loop(0, CHUNK // L, rowmap, 0)

        g1 = pltpu.async_copy(ent_hbm.at[phh], ph_rows, sem)
        g2 = pltpu.async_copy(ent_hbm.at[pth], pt_rows, sem)
        g3 = pltpu.async_copy(ent_hbm.at[nhh], nh_rows, sem)
        g4 = pltpu.async_copy(ent_hbm.at[nth], nt_rows, sem)
        g5 = pltpu.async_copy(rel_hbm.at[prv], r_rows, sem)
        g1.wait(); g2.wait(); g3.wait(); g4.wait(); g5.wait()

        def body(g, carry):
            ds16 = pl.ds(g * L, L)
            rowidx = g * L + lane
            phi = phv[ds16]
            pti = ptv[ds16]
            nhi = nhv[ds16]
            nti = ntv[ds16]
            phb = ((phi >> 7) & 1) * D
            ptb = ((pti >> 7) & 1) * D
            nhb = ((nhi >> 7) & 1) * D
            ntb = ((nti >> 7) & 1) * D
            pacc = jnp.zeros((L,), jnp.float32)
            nacc = jnp.zeros((L,), jnp.float32)
            for d in range(D):
                rcol = (lane + d) & (D - 1)
                r = plsc.load_gather(r_rows, [rowidx, rcol])
                ph = plsc.load_gather(ph_rows, [rowidx, rcol + phb])
                pt = plsc.load_gather(pt_rows, [rowidx, rcol + ptb])
                nh = plsc.load_gather(nh_rows, [rowidx, rcol + nhb])
                nt = plsc.load_gather(nt_rows, [rowidx, rcol + ntb])
                pacc = pacc + jnp.abs(ph + r - pt)
                nacc = nacc + jnp.abs(nh + r - nt)
            pos_v[pl.ds(g * L, L)] = pacc
            neg_v[pl.ds(g * L, L)] = nacc
            return carry

        lax.fori_loop(0, CHUNK // L, body, 0)
        pltpu.sync_copy(pos_v, pos_hbm.at[sl])
        pltpu.sync_copy(neg_v, neg_hbm.at[sl])
        return chunk_carry

    lax.fori_loop(0, NCHUNK, chunk_body, 0)


@jax.jit
def kernel(pos_samples, neg_samples, entity_table, relation_table):
    ph = pos_samples[:, 0].astype(jnp.int32)
    pr = pos_samples[:, 1].astype(jnp.int32)
    pt = pos_samples[:, 2].astype(jnp.int32)
    nh = neg_samples[:, 0].astype(jnp.int32)
    nt = neg_samples[:, 2].astype(jnp.int32)
    entT = entity_table.T
    relp = jnp.pad(relation_table, ((0, 0), (0, W - D)))

    fmt = pl.pallas_call(
        _format_tc,
        grid=(GRID,),
        in_specs=[pl.BlockSpec((D, EB), lambda j: (0, j))],
        out_specs=pl.BlockSpec((EB // 2, W), lambda j: (j, 0)),
        out_shape=jax.ShapeDtypeStruct((FR, W), jnp.float32),
        compiler_params=pltpu.CompilerParams(
            dimension_semantics=("arbitrary",)),
    )(entT)

    mesh = plsc.VectorSubcoreMesh(core_axis_name="c", subcore_axis_name="s")
    params = pltpu.CompilerParams(
        needs_layout_passes=False, use_tc_tiling_on_sc=True)

    score = pl.kernel(
        _score_sc,
        out_type=(
            jax.ShapeDtypeStruct((B,), jnp.float32),
            jax.ShapeDtypeStruct((B,), jnp.float32),
        ),
        mesh=mesh,
        compiler_params=params,
        scratch_types=[
            pltpu.VMEM((CHUNK,), jnp.int32),
            pltpu.VMEM((CHUNK,), jnp.int32),
            pltpu.VMEM((CHUNK,), jnp.int32),
            pltpu.VMEM((CHUNK,), jnp.int32),
            pltpu.VMEM((CHUNK,), jnp.int32),
            pltpu.VMEM((CHUNK,), jnp.int32),
            pltpu.VMEM((CHUNK,), jnp.int32),
            pltpu.VMEM((CHUNK,), jnp.int32),
            pltpu.VMEM((CHUNK,), jnp.int32),
            pltpu.VMEM((CHUNK, W), jnp.float32),
            pltpu.VMEM((CHUNK, W), jnp.float32),
            pltpu.VMEM((CHUNK, W), jnp.float32),
            pltpu.VMEM((CHUNK, W), jnp.float32),
            pltpu.VMEM((CHUNK, W), jnp.float32),
            pltpu.VMEM((CHUNK,), jnp.float32),
            pltpu.VMEM((CHUNK,), jnp.float32),
            pltpu.SemaphoreType.DMA,
        ],
    )
    return score(ph, pr, pt, nh, nt, fmt, relp)


# speedup vs baseline: 2.0037x; 1.7494x over previous
import jax
import jax.numpy as jnp
from jax import lax
from jax.experimental import pallas as pl
from jax.experimental.pallas import tpu as pltpu
from jax.experimental.pallas import tpu_sc as plsc

B = 16384
NE = 1000000
NR = 1000
D = 64
W = 2 * D
L = 16
NC = 2
NS = 16
NW = NC * NS
BPW = B // NW
CHUNK = 128
NCHUNK = BPW // CHUNK

EB = 8192
GRID = (NE + EB - 1) // EB
FR = GRID * (EB // 2)


def _format_tc(x_ref, o_ref):
    for m in range(EB // 256):
        xa = x_ref[:, m * 256:m * 256 + 128]
        xb = x_ref[:, m * 256 + 128:m * 256 + 256]
        x2 = jnp.concatenate([xa, xb], axis=0)
        o_ref[m * 128:(m + 1) * 128, :] = jnp.transpose(x2)


def _score_sc(ph_hbm, pr_hbm, pt_hbm, nh_hbm, nt_hbm, ent_hbm, rel_hbm,
              pos_hbm, neg_hbm,
              phv, prv, ptv, nhv, ntv,
              phh, pth, nhh, nth,
              ph_rows, pt_rows, nh_rows, nt_rows, r_rows,
              pos_v, neg_v, sem):
    wid = lax.axis_index("s") * NC + lax.axis_index("c")
    lane = lax.iota(jnp.int32, L)

    def chunk_body(c, chunk_carry):
        base = wid * BPW + c * CHUNK
        sl = pl.ds(base, CHUNK)
        pltpu.sync_copy(ph_hbm.at[sl], phv)
        pltpu.sync_copy(pr_hbm.at[sl], prv)
        pltpu.sync_copy(pt_hbm.at[sl], ptv)
        pltpu.sync_copy(nh_hbm.at[sl], nhv)
        pltpu.sync_copy(nt_hbm.at[sl], ntv)

        def rowmap(i, carry):
            ds16 = pl.ds(i * L, L)
            phh[ds16] = ((phv[ds16] >> 8) << 7) + (phv[ds16] & 127)
            pth[ds16] = ((ptv[ds16] >> 8) << 7) + (ptv[ds16] & 127)
            nhh[ds16] = ((nhv[ds16] >> 8) << 7) + (nhv[ds16] & 127)
            nth[ds16] = ((ntv[ds16] >> 8) << 7) + (ntv[ds16] & 127)
            return carry

        lax.fori_loop(0, CHUNK // L, rowmap, 0)

        g1 = pltpu.async_copy(ent_hbm.at[phh], ph_rows, sem)
        g2 = pltpu.async_copy(ent_hbm.at[pth], pt_rows, sem)
        g3 = pltpu.async_copy(ent_hbm.at[nhh], nh_rows, sem)
        g4 = pltpu.async_copy(ent_hbm.at[nth], nt_rows, sem)
        g5 = pltpu.async_copy(rel_hbm.at[prv], r_rows, sem)
        g1.wait(); g2.wait(); g3.wait(); g4.wait(); g5.wait()

        def body(g, carry):
            ds16 = pl.ds(g * L, L)
            rowidx = g * L + lane
            phi = phv[ds16]
            pti = ptv[ds16]
            nhi = nhv[ds16]
            nti = ntv[ds16]
            phb = ((phi >> 7) & 1) * D
            ptb = ((pti >> 7) & 1) * D
            nhb = ((nhi >> 7) & 1) * D
            ntb = ((nti >> 7) & 1) * D
            pacc = jnp.zeros((L,), jnp.float32)
            nacc = jnp.zeros((L,), jnp.float32)
            for d in range(D):
                rcol = (lane + d) & (D - 1)
                r = plsc.load_gather(r_rows, [rowidx, rcol])
                ph = plsc.load_gather(ph_rows, [rowidx, rcol + phb])
                pt = plsc.load_gather(pt_rows, [rowidx, rcol + ptb])
                nh = plsc.load_gather(nh_rows, [rowidx, rcol + nhb])
                nt = plsc.load_gather(nt_rows, [rowidx, rcol + ntb])
                pacc = pacc + jnp.abs(ph + r - pt)
                nacc = nacc + jnp.abs(nh + r - nt)
            pos_v[pl.ds(g * L, L)] = pacc
            neg_v[pl.ds(g * L, L)] = nacc
            return carry

        lax.fori_loop(0, CHUNK // L, body, 0)
        pltpu.sync_copy(pos_v, pos_hbm.at[sl])
        pltpu.sync_copy(neg_v, neg_hbm.at[sl])
        return chunk_carry

    lax.fori_loop(0, NCHUNK, chunk_body, 0)


@jax.jit
def kernel(pos_samples, neg_samples, entity_table, relation_table):
    ph = pos_samples[:, 0].astype(jnp.int32)
    pr = pos_samples[:, 1].astype(jnp.int32)
    pt = pos_samples[:, 2].astype(jnp.int32)
    nh = neg_samples[:, 0].astype(jnp.int32)
    nt = neg_samples[:, 2].astype(jnp.int32)
    entT = entity_table.T
    relp = jnp.pad(relation_table, ((0, 0), (0, W - D)))

    fmt = pl.pallas_call(
        _format_tc,
        grid=(GRID,),
        in_specs=[pl.BlockSpec((D, EB), lambda j: (0, j))],
        out_specs=pl.BlockSpec((EB // 2, W), lambda j: (j, 0)),
        out_shape=jax.ShapeDtypeStruct((FR, W), jnp.float32),
        compiler_params=pltpu.CompilerParams(
            dimension_semantics=("arbitrary",)),
    )(entT)

    mesh = plsc.VectorSubcoreMesh(core_axis_name="c", subcore_axis_name="s")
    params = pltpu.CompilerParams(
        needs_layout_passes=False, use_tc_tiling_on_sc=True)

    score = pl.kernel(
        _score_sc,
        out_type=(
            jax.ShapeDtypeStruct((B,), jnp.float32),
            jax.ShapeDtypeStruct((B,), jnp.float32),
        ),
        mesh=mesh,
        compiler_params=params,
        scratch_types=[
            pltpu.VMEM((CHUNK,), jnp.int32),
            pltpu.VMEM((CHUNK,), jnp.int32),
            pltpu.VMEM((CHUNK,), jnp.int32),
            pltpu.VMEM((CHUNK,), jnp.int32),
            pltpu.VMEM((CHUNK,), jnp.int32),
            pltpu.VMEM((CHUNK,), jnp.int32),
            pltpu.VMEM((CHUNK,), jnp.int32),
            pltpu.VMEM((CHUNK,), jnp.int32),
            pltpu.VMEM((CHUNK,), jnp.int32),
            pltpu.VMEM((CHUNK, W), jnp.float32),
            pltpu.VMEM((CHUNK, W), jnp.float32),
            pltpu.VMEM((CHUNK, W), jnp.float32),
            pltpu.VMEM((CHUNK, W), jnp.float32),
            pltpu.VMEM((CHUNK, W), jnp.float32),
            pltpu.VMEM((CHUNK,), jnp.float32),
            pltpu.VMEM((CHUNK,), jnp.float32),
            pltpu.SemaphoreType.DMA,
        ],
    )
    return score(ph, pr, pt, nh, nt, fmt, relp)


# speedup vs baseline: 2.2525x; 1.1241x over previous
import jax
import jax.numpy as jnp
from jax import lax
from jax.experimental import pallas as pl
from jax.experimental.pallas import tpu as pltpu
from jax.experimental.pallas import tpu_sc as plsc

B = 16384
NE = 1000000
NR = 1000
D = 64
W = 2 * D
L = 16
NC = 2
NS = 16
NW = NC * NS
BPW = B // NW
CHUNK = 128
NCHUNK = BPW // CHUNK

EB = 16384
GRID = (NE + EB - 1) // EB
FR = GRID * (EB // 2)


def _format_tc(x_ref, o_ref):
    for m in range(EB // 256):
        xa = x_ref[:, m * 256:m * 256 + 128]
        xb = x_ref[:, m * 256 + 128:m * 256 + 256]
        x2 = jnp.concatenate([xa, xb], axis=0)
        o_ref[m * 128:(m + 1) * 128, :] = jnp.transpose(x2)


def _score_sc(ph_hbm, pr_hbm, pt_hbm, nh_hbm, nt_hbm, ent_hbm, rel_hbm,
              pos_hbm, neg_hbm,
              phv, prv, ptv, nhv, ntv,
              phh, pth, nhh, nth,
              ph_rows, pt_rows, nh_rows, nt_rows, r_rows,
              pos_v, neg_v, sem):
    wid = lax.axis_index("s") * NC + lax.axis_index("c")
    lane = lax.iota(jnp.int32, L)

    def chunk_body(c, chunk_carry):
        base = wid * BPW + c * CHUNK
        sl = pl.ds(base, CHUNK)
        pltpu.sync_copy(ph_hbm.at[sl], phv)
        pltpu.sync_copy(pr_hbm.at[sl], prv)
        pltpu.sync_copy(pt_hbm.at[sl], ptv)
        pltpu.sync_copy(nh_hbm.at[sl], nhv)
        pltpu.sync_copy(nt_hbm.at[sl], ntv)

        def rowmap(i, carry):
            ds16 = pl.ds(i * L, L)
            phh[ds16] = ((phv[ds16] >> 8) << 7) + (phv[ds16] & 127)
            pth[ds16] = ((ptv[ds16] >> 8) << 7) + (ptv[ds16] & 127)
            nhh[ds16] = ((nhv[ds16] >> 8) << 7) + (nhv[ds16] & 127)
            nth[ds16] = ((ntv[ds16] >> 8) << 7) + (ntv[ds16] & 127)
            return carry

        lax.fori_loop(0, CHUNK // L, rowmap, 0)

        g1 = pltpu.async_copy(ent_hbm.at[phh], ph_rows, sem)
        g2 = pltpu.async_copy(ent_hbm.at[pth], pt_rows, sem)
        g3 = pltpu.async_copy(ent_hbm.at[nhh], nh_rows, sem)
        g4 = pltpu.async_copy(ent_hbm.at[nth], nt_rows, sem)
        g5 = pltpu.async_copy(rel_hbm.at[prv], r_rows, sem)
        g1.wait(); g2.wait(); g3.wait(); g4.wait(); g5.wait()

        def body(g, carry):
            ds16 = pl.ds(g * L, L)
            rowidx = g * L + lane
            phi = phv[ds16]
            pti = ptv[ds16]
            nhi = nhv[ds16]
            nti = ntv[ds16]
            phb = ((phi >> 7) & 1) * D
            ptb = ((pti >> 7) & 1) * D
            nhb = ((nhi >> 7) & 1) * D
            ntb = ((nti >> 7) & 1) * D
            pacc = jnp.zeros((L,), jnp.float32)
            nacc = jnp.zeros((L,), jnp.float32)
            for d in range(D):
                rcol = (lane + d) & (D - 1)
                r = plsc.load_gather(r_rows, [rowidx, rcol])
                ph = plsc.load_gather(ph_rows, [rowidx, rcol + phb])
                pt = plsc.load_gather(pt_rows, [rowidx, rcol + ptb])
                nh = plsc.load_gather(nh_rows, [rowidx, rcol + nhb])
                nt = plsc.load_gather(nt_rows, [rowidx, rcol + ntb])
                pacc = pacc + jnp.abs(ph + r - pt)
                nacc = nacc + jnp.abs(nh + r - nt)
            pos_v[pl.ds(g * L, L)] = pacc
            neg_v[pl.ds(g * L, L)] = nacc
            return carry

        lax.fori_loop(0, CHUNK // L, body, 0)
        pltpu.sync_copy(pos_v, pos_hbm.at[sl])
        pltpu.sync_copy(neg_v, neg_hbm.at[sl])
        return chunk_carry

    lax.fori_loop(0, NCHUNK, chunk_body, 0)


@jax.jit
def kernel(pos_samples, neg_samples, entity_table, relation_table):
    ph = pos_samples[:, 0].astype(jnp.int32)
    pr = pos_samples[:, 1].astype(jnp.int32)
    pt = pos_samples[:, 2].astype(jnp.int32)
    nh = neg_samples[:, 0].astype(jnp.int32)
    nt = neg_samples[:, 2].astype(jnp.int32)
    entT = entity_table.T
    relp = jnp.pad(relation_table, ((0, 0), (0, W - D)))

    fmt = pl.pallas_call(
        _format_tc,
        grid=(GRID,),
        in_specs=[pl.BlockSpec((D, EB), lambda j: (0, j))],
        out_specs=pl.BlockSpec((EB // 2, W), lambda j: (j, 0)),
        out_shape=jax.ShapeDtypeStruct((FR, W), jnp.float32),
        compiler_params=pltpu.CompilerParams(
            dimension_semantics=("arbitrary",)),
    )(entT)

    mesh = plsc.VectorSubcoreMesh(core_axis_name="c", subcore_axis_name="s")
    params = pltpu.CompilerParams(
        needs_layout_passes=False, use_tc_tiling_on_sc=True)

    score = pl.kernel(
        _score_sc,
        out_type=(
            jax.ShapeDtypeStruct((B,), jnp.float32),
            jax.ShapeDtypeStruct((B,), jnp.float32),
        ),
        mesh=mesh,
        compiler_params=params,
        scratch_types=[
            pltpu.VMEM((CHUNK,), jnp.int32),
            pltpu.VMEM((CHUNK,), jnp.int32),
            pltpu.VMEM((CHUNK,), jnp.int32),
            pltpu.VMEM((CHUNK,), jnp.int32),
            pltpu.VMEM((CHUNK,), jnp.int32),
            pltpu.VMEM((CHUNK,), jnp.int32),
            pltpu.VMEM((CHUNK,), jnp.int32),
            pltpu.VMEM((CHUNK,), jnp.int32),
            pltpu.VMEM((CHUNK,), jnp.int32),
            pltpu.VMEM((CHUNK, W), jnp.float32),
            pltpu.VMEM((CHUNK, W), jnp.float32),
            pltpu.VMEM((CHUNK, W), jnp.float32),
            pltpu.VMEM((CHUNK, W), jnp.float32),
            pltpu.VMEM((CHUNK, W), jnp.float32),
            pltpu.VMEM((CHUNK,), jnp.float32),
            pltpu.VMEM((CHUNK,), jnp.float32),
            pltpu.SemaphoreType.DMA,
        ],
    )
    return score(ph, pr, pt, nh, nt, fmt, relp)


# speedup vs baseline: 2.3053x; 1.0235x over previous
import jax
import jax.numpy as jnp
from jax import lax
from jax.experimental import pallas as pl
from jax.experimental.pallas import tpu as pltpu
from jax.experimental.pallas import tpu_sc as plsc

B = 16384
NE = 1000000
NR = 1000
D = 64
W = 2 * D
L = 16
NC = 2
NS = 16
NW = NC * NS
BPW = B // NW
CHUNK = 128
NCHUNK = BPW // CHUNK

EB = 32768
GRID = (NE + EB - 1) // EB
FR = GRID * (EB // 2)


def _format_tc(x_ref, o_ref):
    for m in range(EB // 256):
        xa = x_ref[:, m * 256:m * 256 + 128]
        xb = x_ref[:, m * 256 + 128:m * 256 + 256]
        x2 = jnp.concatenate([xa, xb], axis=0)
        o_ref[m * 128:(m + 1) * 128, :] = jnp.transpose(x2)


def _score_sc(ph_hbm, pr_hbm, pt_hbm, nh_hbm, nt_hbm, ent_hbm, rel_hbm,
              pos_hbm, neg_hbm,
              phv, prv, ptv, nhv, ntv,
              phh, pth, nhh, nth,
              ph_rows, pt_rows, nh_rows, nt_rows, r_rows,
              pos_v, neg_v, sem):
    wid = lax.axis_index("s") * NC + lax.axis_index("c")
    lane = lax.iota(jnp.int32, L)

    def chunk_body(c, chunk_carry):
        base = wid * BPW + c * CHUNK
        sl = pl.ds(base, CHUNK)
        pltpu.sync_copy(ph_hbm.at[sl], phv)
        pltpu.sync_copy(pr_hbm.at[sl], prv)
        pltpu.sync_copy(pt_hbm.at[sl], ptv)
        pltpu.sync_copy(nh_hbm.at[sl], nhv)
        pltpu.sync_copy(nt_hbm.at[sl], ntv)

        def rowmap(i, carry):
            ds16 = pl.ds(i * L, L)
            phh[ds16] = ((phv[ds16] >> 8) << 7) + (phv[ds16] & 127)
            pth[ds16] = ((ptv[ds16] >> 8) << 7) + (ptv[ds16] & 127)
            nhh[ds16] = ((nhv[ds16] >> 8) << 7) + (nhv[ds16] & 127)
            nth[ds16] = ((ntv[ds16] >> 8) << 7) + (ntv[ds16] & 127)
            return carry

        lax.fori_loop(0, CHUNK // L, rowmap, 0)

        g1 = pltpu.async_copy(ent_hbm.at[phh], ph_rows, sem)
        g2 = pltpu.async_copy(ent_hbm.at[pth], pt_rows, sem)
        g3 = pltpu.async_copy(ent_hbm.at[nhh], nh_rows, sem)
        g4 = pltpu.async_copy(ent_hbm.at[nth], nt_rows, sem)
        g5 = pltpu.async_copy(rel_hbm.at[prv], r_rows, sem)
        g1.wait(); g2.wait(); g3.wait(); g4.wait(); g5.wait()

        def body(g, carry):
            ds16 = pl.ds(g * L, L)
            rowidx = g * L + lane
            phi = phv[ds16]
            pti = ptv[ds16]
            nhi = nhv[ds16]
            nti = ntv[ds16]
            phb = ((phi >> 7) & 1) * D
            ptb = ((pti >> 7) & 1) * D
            nhb = ((nhi >> 7) & 1) * D
            ntb = ((nti >> 7) & 1) * D
            pacc = jnp.zeros((L,), jnp.float32)
            nacc = jnp.zeros((L,), jnp.float32)
            for d in range(D):
                rcol = (lane + d) & (D - 1)
                r = plsc.load_gather(r_rows, [rowidx, rcol])
                ph = plsc.load_gather(ph_rows, [rowidx, rcol + phb])
                pt = plsc.load_gather(pt_rows, [rowidx, rcol + ptb])
                nh = plsc.load_gather(nh_rows, [rowidx, rcol + nhb])
                nt = plsc.load_gather(nt_rows, [rowidx, rcol + ntb])
                pacc = pacc + jnp.abs(ph + r - pt)
                nacc = nacc + jnp.abs(nh + r - nt)
            pos_v[pl.ds(g * L, L)] = pacc
            neg_v[pl.ds(g * L, L)] = nacc
            return carry

        lax.fori_loop(0, CHUNK // L, body, 0)
        pltpu.sync_copy(pos_v, pos_hbm.at[sl])
        pltpu.sync_copy(neg_v, neg_hbm.at[sl])
        return chunk_carry

    lax.fori_loop(0, NCHUNK, chunk_body, 0)


@jax.jit
def kernel(pos_samples, neg_samples, entity_table, relation_table):
    ph = pos_samples[:, 0].astype(jnp.int32)
    pr = pos_samples[:, 1].astype(jnp.int32)
    pt = pos_samples[:, 2].astype(jnp.int32)
    nh = neg_samples[:, 0].astype(jnp.int32)
    nt = neg_samples[:, 2].astype(jnp.int32)
    entT = entity_table.T
    relp = jnp.pad(relation_table, ((0, 0), (0, W - D)))

    fmt = pl.pallas_call(
        _format_tc,
        grid=(GRID,),
        in_specs=[pl.BlockSpec((D, EB), lambda j: (0, j))],
        out_specs=pl.BlockSpec((EB // 2, W), lambda j: (j, 0)),
        out_shape=jax.ShapeDtypeStruct((FR, W), jnp.float32),
        compiler_params=pltpu.CompilerParams(
            dimension_semantics=("arbitrary",)),
    )(entT)

    mesh = plsc.VectorSubcoreMesh(core_axis_name="c", subcore_axis_name="s")
    params = pltpu.CompilerParams(
        needs_layout_passes=False, use_tc_tiling_on_sc=True)

    score = pl.kernel(
        _score_sc,
        out_type=(
            jax.ShapeDtypeStruct((B,), jnp.float32),
            jax.ShapeDtypeStruct((B,), jnp.float32),
        ),
        mesh=mesh,
        compiler_params=params,
        scratch_types=[
            pltpu.VMEM((CHUNK,), jnp.int32),
            pltpu.VMEM((CHUNK,), jnp.int32),
            pltpu.VMEM((CHUNK,), jnp.int32),
            pltpu.VMEM((CHUNK,), jnp.int32),
            pltpu.VMEM((CHUNK,), jnp.int32),
            pltpu.VMEM((CHUNK,), jnp.int32),
            pltpu.VMEM((CHUNK,), jnp.int32),
            pltpu.VMEM((CHUNK,), jnp.int32),
            pltpu.VMEM((CHUNK,), jnp.int32),
            pltpu.VMEM((CHUNK, W), jnp.float32),
            pltpu.VMEM((CHUNK, W), jnp.float32),
            pltpu.VMEM((CHUNK, W), jnp.float32),
            pltpu.VMEM((CHUNK, W), jnp.float32),
            pltpu.VMEM((CHUNK, W), jnp.float32),
            pltpu.VMEM((CHUNK,), jnp.float32),
            pltpu.VMEM((CHUNK,), jnp.float32),
            pltpu.SemaphoreType.DMA,
        ],
    )
    return score(ph, pr, pt, nh, nt, fmt, relp)
